# baseline (device time: 19233 ns/iter reference)
import jax
import jax.numpy as jnp
from jax import lax
from jax.experimental import pallas as pl
from jax.experimental.pallas import tpu as pltpu


def kernel(partial, gamma):
    xs, m2, d = partial.shape
    m = m2 // 2
    q = m // 2
    h = q // 2

    x2 = partial.reshape(m2, d)
    g2 = gamma.reshape(1, d)

    def body(x_ref, g_ref, out_ref, comm_ref, sems):
        my_x = lax.axis_index("x")
        my_y = lax.axis_index("y")
        my_z = lax.axis_index("z")
        s = (my_y + my_z) % 2

        barrier = pltpu.get_barrier_semaphore()
        for dev in (
            (1 - my_x, my_y, my_z),
            (my_x, 1 - my_y, my_z),
            (my_x, my_y, 1 - my_z),
        ):
            pl.semaphore_signal(
                barrier, inc=1, device_id=dev,
                device_id_type=pl.DeviceIdType.MESH,
            )
        pl.semaphore_wait(barrier, 3)

        rdma1 = pltpu.make_async_remote_copy(
            src_ref=x_ref.at[pl.ds((1 - my_x) * m + s * q, q), :],
            dst_ref=comm_ref,
            send_sem=sems.at[0],
            recv_sem=sems.at[1],
            device_id=(1 - my_x, my_y, my_z),
            device_id_type=pl.DeviceIdType.MESH,
        )
        rdma1.start()
        rdma1.wait()

        red = x_ref[pl.ds(my_x * m + s * q, q), :] + comm_ref[...]
        rms = jnp.sqrt(jnp.mean(red * red, axis=-1, keepdims=True) + 1e-6)
        out_ref[pl.ds(s * q, q), :] = red / rms * g_ref[...]

        rdma2y = pltpu.make_async_remote_copy(
            src_ref=out_ref.at[pl.ds(s * q, h), :],
            dst_ref=out_ref.at[pl.ds(s * q, h), :],
            send_sem=sems.at[2],
            recv_sem=sems.at[3],
            device_id=(my_x, 1 - my_y, my_z),
            device_id_type=pl.DeviceIdType.MESH,
        )
        rdma2z = pltpu.make_async_remote_copy(
            src_ref=out_ref.at[pl.ds(s * q + h, h), :],
            dst_ref=out_ref.at[pl.ds(s * q + h, h), :],
            send_sem=sems.at[4],
            recv_sem=sems.at[5],
            device_id=(my_x, my_y, 1 - my_z),
            device_id_type=pl.DeviceIdType.MESH,
        )
        rdma2y.start()
        rdma2z.start()
        rdma2y.wait()
        rdma2z.wait()

    return pl.pallas_call(
        body,
        out_shape=jax.ShapeDtypeStruct((m, d), jnp.float32),
        in_specs=[
            pl.BlockSpec(memory_space=pltpu.VMEM),
            pl.BlockSpec(memory_space=pltpu.VMEM),
        ],
        out_specs=pl.BlockSpec(memory_space=pltpu.VMEM),
        scratch_shapes=[
            pltpu.VMEM((q, d), jnp.float32),
            pltpu.SemaphoreType.DMA((6,)),
        ],
        compiler_params=pltpu.CompilerParams(collective_id=0),
    )(x2, g2)


# device time: 17873 ns/iter; 1.0761x vs baseline; 1.0761x over previous
import jax
import jax.numpy as jnp
from jax import lax
from jax.experimental import pallas as pl
from jax.experimental.pallas import tpu as pltpu


NC = 4


def kernel(partial, gamma):
    xs, m2, d = partial.shape
    m = m2 // 2
    q = m // 2
    ch = q // NC

    x2 = partial.reshape(m2, d)
    g2 = gamma.reshape(1, d)

    def body(x_ref, g_ref, out_ref, comm_ref, p1_send, p1_recv, p2_send, p2_recv):
        my_x = lax.axis_index("x")
        my_y = lax.axis_index("y")
        my_z = lax.axis_index("z")
        s = (my_y + my_z) % 2

        x_part = (1 - my_x, my_y, my_z)
        y_part = (my_x, 1 - my_y, my_z)
        z_part = (my_x, my_y, 1 - my_z)

        barrier = pltpu.get_barrier_semaphore()
        for dev in (x_part, y_part, z_part):
            pl.semaphore_signal(
                barrier, inc=1, device_id=dev,
                device_id_type=pl.DeviceIdType.MESH,
            )
        pl.semaphore_wait(barrier, 3)

        p1 = []
        for c in range(NC):
            r = pltpu.make_async_remote_copy(
                src_ref=x_ref.at[pl.ds((1 - my_x) * m + s * q + c * ch, ch), :],
                dst_ref=comm_ref.at[pl.ds(c * ch, ch), :],
                send_sem=p1_send.at[c],
                recv_sem=p1_recv.at[c],
                device_id=x_part,
                device_id_type=pl.DeviceIdType.MESH,
            )
            r.start()
            p1.append(r)

        p2 = []
        for c in range(NC):
            p1[c].wait_recv()
            red = x_ref[pl.ds(my_x * m + s * q + c * ch, ch), :] + \
                comm_ref[pl.ds(c * ch, ch), :]
            rms = jnp.sqrt(jnp.mean(red * red, axis=-1, keepdims=True) + 1e-6)
            out_ref[pl.ds(s * q + c * ch, ch), :] = red / rms * g_ref[...]
            r = pltpu.make_async_remote_copy(
                src_ref=out_ref.at[pl.ds(s * q + c * ch, ch), :],
                dst_ref=out_ref.at[pl.ds(s * q + c * ch, ch), :],
                send_sem=p2_send.at[c],
                recv_sem=p2_recv.at[c],
                device_id=y_part if c % 2 == 0 else z_part,
                device_id_type=pl.DeviceIdType.MESH,
            )
            r.start()
            p2.append(r)

        for c in range(NC):
            p2[c].wait_recv()
        for c in range(NC):
            p1[c].wait_send()
            p2[c].wait_send()

    return pl.pallas_call(
        body,
        out_shape=jax.ShapeDtypeStruct((m, d), jnp.float32),
        in_specs=[
            pl.BlockSpec(memory_space=pltpu.VMEM),
            pl.BlockSpec(memory_space=pltpu.VMEM),
        ],
        out_specs=pl.BlockSpec(memory_space=pltpu.VMEM),
        scratch_shapes=[
            pltpu.VMEM((q, d), jnp.float32),
            pltpu.SemaphoreType.DMA((NC,)),
            pltpu.SemaphoreType.DMA((NC,)),
            pltpu.SemaphoreType.DMA((NC,)),
            pltpu.SemaphoreType.DMA((NC,)),
        ],
        compiler_params=pltpu.CompilerParams(collective_id=0),
    )(x2, g2)


# device time: 17321 ns/iter; 1.1104x vs baseline; 1.0319x over previous
import jax
import jax.numpy as jnp
from jax import lax
from jax.experimental import pallas as pl
from jax.experimental.pallas import tpu as pltpu


NC = 8


def kernel(partial, gamma):
    xs, m2, d = partial.shape
    m = m2 // 2
    q = m // 2
    ch = q // NC

    x2 = partial.reshape(m2, d)
    g2 = gamma.reshape(1, d)

    def body(x_ref, g_ref, out_ref, comm_ref, p1_send, p1_recv, p2_send, p2_recv):
        my_x = lax.axis_index("x")
        my_y = lax.axis_index("y")
        my_z = lax.axis_index("z")
        s = (my_y + my_z) % 2

        x_part = (1 - my_x, my_y, my_z)
        y_part = (my_x, 1 - my_y, my_z)
        z_part = (my_x, my_y, 1 - my_z)

        barrier = pltpu.get_barrier_semaphore()
        for dev in (x_part, y_part, z_part):
            pl.semaphore_signal(
                barrier, inc=1, device_id=dev,
                device_id_type=pl.DeviceIdType.MESH,
            )
        pl.semaphore_wait(barrier, 3)

        p1 = []
        for c in range(NC):
            r = pltpu.make_async_remote_copy(
                src_ref=x_ref.at[pl.ds((1 - my_x) * m + s * q + c * ch, ch), :],
                dst_ref=comm_ref.at[pl.ds(c * ch, ch), :],
                send_sem=p1_send.at[c],
                recv_sem=p1_recv.at[c],
                device_id=x_part,
                device_id_type=pl.DeviceIdType.MESH,
            )
            r.start()
            p1.append(r)

        p2 = []
        for c in range(NC):
            p1[c].wait_recv()
            red = x_ref[pl.ds(my_x * m + s * q + c * ch, ch), :] + \
                comm_ref[pl.ds(c * ch, ch), :]
            rms = jnp.sqrt(jnp.mean(red * red, axis=-1, keepdims=True) + 1e-6)
            out_ref[pl.ds(s * q + c * ch, ch), :] = red / rms * g_ref[...]
            r = pltpu.make_async_remote_copy(
                src_ref=out_ref.at[pl.ds(s * q + c * ch, ch), :],
                dst_ref=out_ref.at[pl.ds(s * q + c * ch, ch), :],
                send_sem=p2_send.at[c],
                recv_sem=p2_recv.at[c],
                device_id=y_part if c % 2 == 0 else z_part,
                device_id_type=pl.DeviceIdType.MESH,
            )
            r.start()
            p2.append(r)

        for c in range(NC):
            p2[c].wait_recv()
        for c in range(NC):
            p1[c].wait_send()
            p2[c].wait_send()

    return pl.pallas_call(
        body,
        out_shape=jax.ShapeDtypeStruct((m, d), jnp.float32),
        in_specs=[
            pl.BlockSpec(memory_space=pltpu.VMEM),
            pl.BlockSpec(memory_space=pltpu.VMEM),
        ],
        out_specs=pl.BlockSpec(memory_space=pltpu.VMEM),
        scratch_shapes=[
            pltpu.VMEM((q, d), jnp.float32),
            pltpu.SemaphoreType.DMA((NC,)),
            pltpu.SemaphoreType.DMA((NC,)),
            pltpu.SemaphoreType.DMA((NC,)),
            pltpu.SemaphoreType.DMA((NC,)),
        ],
        compiler_params=pltpu.CompilerParams(collective_id=0),
    )(x2, g2)


# device time: 16386 ns/iter; 1.1737x vs baseline; 1.0571x over previous
import jax
import jax.numpy as jnp
from jax import lax
from jax.experimental import pallas as pl
from jax.experimental.pallas import tpu as pltpu

TAU = 24

_SCHED = [
    (0, 32, "y"),
    (128, 32, "z"),
    (32, 32, "y"),
    (160, 32, "z"),
    (64, 40, "y"),
    (192, 40, "z"),
    (104, TAU, "own"),
    (232, TAU, "own"),
    (104, TAU, "extra"),
    (232, TAU, "extra"),
]
_N_FWD = sum(1 for _, _, r in _SCHED if r in ("y", "z"))


def kernel(partial, gamma):
    xs, m2, d = partial.shape
    m = m2 // 2
    q = m // 2

    n_comm = sum(n for _, n, _ in _SCHED)
    x2 = partial.reshape(m2, d)
    g2 = gamma.reshape(1, d)

    def body(x_ref, g_ref, out_ref, comm_ref, p1_send, p1_recv, p2_send, p2_recv):
        my_x = lax.axis_index("x")
        my_y = lax.axis_index("y")
        my_z = lax.axis_index("z")
        s = (my_y + my_z) % 2

        x_part = (1 - my_x, my_y, my_z)
        y_part = (my_x, 1 - my_y, my_z)
        z_part = (my_x, my_y, 1 - my_z)

        def half_off(off, role):
            base = (1 - s) * q if role == "extra" else s * q
            return base + off

        barrier = pltpu.get_barrier_semaphore()
        for dev in (x_part, y_part, z_part):
            pl.semaphore_signal(
                barrier, inc=1, device_id=dev,
                device_id_type=pl.DeviceIdType.MESH,
            )
        pl.semaphore_wait(barrier, 3)

        p1 = []
        coff = 0
        for c, (off, n, role) in enumerate(_SCHED):
            ho = half_off(off, role)
            r = pltpu.make_async_remote_copy(
                src_ref=x_ref.at[pl.ds((1 - my_x) * m + ho, n), :],
                dst_ref=comm_ref.at[pl.ds(coff, n), :],
                send_sem=p1_send.at[c],
                recv_sem=p1_recv.at[c],
                device_id=x_part,
                device_id_type=pl.DeviceIdType.MESH,
            )
            r.start()
            p1.append((r, coff))
            coff += n

        p2 = []
        for c, (off, n, role) in enumerate(_SCHED):
            rdma, coff = p1[c]
            rdma.wait_recv()
            ho = half_off(off, role)
            red = x_ref[pl.ds(my_x * m + ho, n), :] + comm_ref[pl.ds(coff, n), :]
            rms = jnp.sqrt(jnp.mean(red * red, axis=-1, keepdims=True) + 1e-6)
            out_ref[pl.ds(ho, n), :] = red / rms * g_ref[...]
            if role in ("y", "z"):
                fc = len(p2)
                r = pltpu.make_async_remote_copy(
                    src_ref=out_ref.at[pl.ds(ho, n), :],
                    dst_ref=out_ref.at[pl.ds(ho, n), :],
                    send_sem=p2_send.at[fc],
                    recv_sem=p2_recv.at[fc],
                    device_id=y_part if role == "y" else z_part,
                    device_id_type=pl.DeviceIdType.MESH,
                )
                r.start()
                p2.append(r)

        for r in p2:
            r.wait_recv()
        for r, _ in p1:
            r.wait_send()
        for r in p2:
            r.wait_send()

    return pl.pallas_call(
        body,
        out_shape=jax.ShapeDtypeStruct((m, d), jnp.float32),
        in_specs=[
            pl.BlockSpec(memory_space=pltpu.VMEM),
            pl.BlockSpec(memory_space=pltpu.VMEM),
        ],
        out_specs=pl.BlockSpec(memory_space=pltpu.VMEM),
        scratch_shapes=[
            pltpu.VMEM((n_comm, d), jnp.float32),
            pltpu.SemaphoreType.DMA((len(_SCHED),)),
            pltpu.SemaphoreType.DMA((len(_SCHED),)),
            pltpu.SemaphoreType.DMA((_N_FWD,)),
            pltpu.SemaphoreType.DMA((_N_FWD,)),
        ],
        compiler_params=pltpu.CompilerParams(collective_id=0),
    )(x2, g2)


# device time: 16067 ns/iter; 1.1970x vs baseline; 1.0199x over previous
import jax
import jax.numpy as jnp
from jax import lax
from jax.experimental import pallas as pl
from jax.experimental.pallas import tpu as pltpu

TAU = 24

_SCHED = [
    (0, 40, "y"),
    (128, 40, "z"),
    (40, 40, "y"),
    (168, 40, "z"),
    (80, 24, "y"),
    (208, 24, "z"),
    (104, TAU, "own"),
    (232, TAU, "own"),
    (104, TAU, "extra"),
    (232, TAU, "extra"),
]
_N_FWD = sum(1 for _, _, r in _SCHED if r in ("y", "z"))


def kernel(partial, gamma):
    xs, m2, d = partial.shape
    m = m2 // 2
    q = m // 2

    n_comm = sum(n for _, n, _ in _SCHED)
    x2 = partial.reshape(m2, d)
    g2 = gamma.reshape(1, d)

    def body(x_ref, g_ref, out_ref, comm_ref, p1_send, p1_recv, p2_send, p2_recv):
        my_x = lax.axis_index("x")
        my_y = lax.axis_index("y")
        my_z = lax.axis_index("z")
        s = (my_y + my_z) % 2

        x_part = (1 - my_x, my_y, my_z)
        y_part = (my_x, 1 - my_y, my_z)
        z_part = (my_x, my_y, 1 - my_z)

        def half_off(off, role):
            base = (1 - s) * q if role == "extra" else s * q
            return base + off

        barrier = pltpu.get_barrier_semaphore()
        for dev in (x_part, y_part, z_part):
            pl.semaphore_signal(
                barrier, inc=1, device_id=dev,
                device_id_type=pl.DeviceIdType.MESH,
            )
        pl.semaphore_wait(barrier, 3)

        p1 = []
        coff = 0
        for c, (off, n, role) in enumerate(_SCHED):
            ho = half_off(off, role)
            r = pltpu.make_async_remote_copy(
                src_ref=x_ref.at[pl.ds((1 - my_x) * m + ho, n), :],
                dst_ref=comm_ref.at[pl.ds(coff, n), :],
                send_sem=p1_send.at[c],
                recv_sem=p1_recv.at[c],
                device_id=x_part,
                device_id_type=pl.DeviceIdType.MESH,
            )
            r.start()
            p1.append((r, coff))
            coff += n

        p2 = []
        for c, (off, n, role) in enumerate(_SCHED):
            rdma, coff = p1[c]
            rdma.wait_recv()
            ho = half_off(off, role)
            red = x_ref[pl.ds(my_x * m + ho, n), :] + comm_ref[pl.ds(coff, n), :]
            rms = jnp.sqrt(jnp.mean(red * red, axis=-1, keepdims=True) + 1e-6)
            out_ref[pl.ds(ho, n), :] = red / rms * g_ref[...]
            if role in ("y", "z"):
                fc = len(p2)
                r = pltpu.make_async_remote_copy(
                    src_ref=out_ref.at[pl.ds(ho, n), :],
                    dst_ref=out_ref.at[pl.ds(ho, n), :],
                    send_sem=p2_send.at[fc],
                    recv_sem=p2_recv.at[fc],
                    device_id=y_part if role == "y" else z_part,
                    device_id_type=pl.DeviceIdType.MESH,
                )
                r.start()
                p2.append(r)

        for r in p2:
            r.wait_recv()
        for r, _ in p1:
            r.wait_send()
        for r in p2:
            r.wait_send()

    return pl.pallas_call(
        body,
        out_shape=jax.ShapeDtypeStruct((m, d), jnp.float32),
        in_specs=[
            pl.BlockSpec(memory_space=pltpu.VMEM),
            pl.BlockSpec(memory_space=pltpu.VMEM),
        ],
        out_specs=pl.BlockSpec(memory_space=pltpu.VMEM),
        scratch_shapes=[
            pltpu.VMEM((n_comm, d), jnp.float32),
            pltpu.SemaphoreType.DMA((len(_SCHED),)),
            pltpu.SemaphoreType.DMA((len(_SCHED),)),
            pltpu.SemaphoreType.DMA((_N_FWD,)),
            pltpu.SemaphoreType.DMA((_N_FWD,)),
        ],
        compiler_params=pltpu.CompilerParams(collective_id=0),
    )(x2, g2)


# device time: 16047 ns/iter; 1.1985x vs baseline; 1.0012x over previous
import jax
import jax.numpy as jnp
from jax import lax
from jax.experimental import pallas as pl
from jax.experimental.pallas import tpu as pltpu

TAU = 24

_SCHED = [
    (0, 40, "y"),
    (128, 40, "z"),
    (40, 40, "y"),
    (168, 40, "z"),
    (80, 24, "y"),
    (208, 24, "z"),
    (104, TAU, "own"),
    (232, TAU, "own"),
    (104, TAU, "extra"),
    (232, TAU, "extra"),
]
_N_FWD = sum(1 for _, _, r in _SCHED if r in ("y", "z"))


def kernel(partial, gamma):
    xs, m2, d = partial.shape
    m = m2 // 2
    q = m // 2

    n_comm = sum(n for _, n, _ in _SCHED)
    g2 = gamma.reshape(1, d)

    def body(x_ref, g_ref, out_ref, comm_ref, p1_send, p1_recv, p2_send, p2_recv):
        my_x = lax.axis_index("x")
        my_y = lax.axis_index("y")
        my_z = lax.axis_index("z")
        s = (my_y + my_z) % 2

        x_part = (1 - my_x, my_y, my_z)
        y_part = (my_x, 1 - my_y, my_z)
        z_part = (my_x, my_y, 1 - my_z)

        def half_off(off, role):
            base = (1 - s) * q if role == "extra" else s * q
            return base + off

        barrier = pltpu.get_barrier_semaphore()
        for dev in (x_part, y_part, z_part):
            pl.semaphore_signal(
                barrier, inc=1, device_id=dev,
                device_id_type=pl.DeviceIdType.MESH,
            )
        pl.semaphore_wait(barrier, 3)

        p1 = []
        coff = 0
        for c, (off, n, role) in enumerate(_SCHED):
            ho = half_off(off, role)
            r = pltpu.make_async_remote_copy(
                src_ref=x_ref.at[0, pl.ds((1 - my_x) * m + ho, n), :],
                dst_ref=comm_ref.at[pl.ds(coff, n), :],
                send_sem=p1_send.at[c],
                recv_sem=p1_recv.at[c],
                device_id=x_part,
                device_id_type=pl.DeviceIdType.MESH,
            )
            r.start()
            p1.append((r, coff))
            coff += n

        p2 = []
        for c, (off, n, role) in enumerate(_SCHED):
            rdma, coff = p1[c]
            rdma.wait_recv()
            ho = half_off(off, role)
            red = x_ref[0, pl.ds(my_x * m + ho, n), :] + comm_ref[pl.ds(coff, n), :]
            rinv = lax.rsqrt(jnp.mean(red * red, axis=-1, keepdims=True) + 1e-6)
            out_ref[pl.ds(ho, n), :] = red * rinv * g_ref[...]
            if role in ("y", "z"):
                fc = len(p2)
                r = pltpu.make_async_remote_copy(
                    src_ref=out_ref.at[pl.ds(ho, n), :],
                    dst_ref=out_ref.at[pl.ds(ho, n), :],
                    send_sem=p2_send.at[fc],
                    recv_sem=p2_recv.at[fc],
                    device_id=y_part if role == "y" else z_part,
                    device_id_type=pl.DeviceIdType.MESH,
                )
                r.start()
                p2.append(r)

        for r in p2:
            r.wait_recv()
        for r, _ in p1:
            r.wait_send()
        for r in p2:
            r.wait_send()

    return pl.pallas_call(
        body,
        out_shape=jax.ShapeDtypeStruct((m, d), jnp.float32),
        in_specs=[
            pl.BlockSpec(memory_space=pltpu.VMEM),
            pl.BlockSpec(memory_space=pltpu.VMEM),
        ],
        out_specs=pl.BlockSpec(memory_space=pltpu.VMEM),
        scratch_shapes=[
            pltpu.VMEM((n_comm, d), jnp.float32),
            pltpu.SemaphoreType.DMA((len(_SCHED),)),
            pltpu.SemaphoreType.DMA((len(_SCHED),)),
            pltpu.SemaphoreType.DMA((_N_FWD,)),
            pltpu.SemaphoreType.DMA((_N_FWD,)),
        ],
        compiler_params=pltpu.CompilerParams(collective_id=0),
    )(partial, g2)
